# Initial kernel scaffold; baseline (speedup 1.0000x reference)
import jax
import jax.numpy as jnp
from jax.experimental import pallas as pl


def kernel(x, edge_index, W1, b1, W2, b2):
    raise NotImplementedError("write your pallas kernel here")


# speedup vs baseline: 9.2283x; 9.2283x over previous
import functools

import jax
import jax.numpy as jnp
from jax import lax
from jax.experimental import pallas as pl
from jax.experimental.pallas import tpu as pltpu
from jax.experimental.pallas import tpu_sc as plsc

N = 10000
E = 160000
IN_CH = 256
HID = 1024
OUT = 128

NPAD = 10240
HALF = 5120
ACC_ROWS = HALF + 16
E_PAD = 163840
K = 128
ROWS2D = E_PAD // K
F = 128


def _mesh():
    return plsc.VectorSubcoreMesh(core_axis_name="c", subcore_axis_name="s")


def _deg_sc(col2d):
    nch = ROWS2D // 32
    seg = NPAD // 16
    grp = 8

    @functools.partial(
        pl.kernel,
        out_type=jax.ShapeDtypeStruct((2, NPAD), jnp.float32),
        mesh=_mesh(),
        scratch_types=[
            pltpu.VMEM((nch, K), jnp.int32),
            pltpu.VMEM((K,), jnp.float32),
            pltpu.VMEM((seg,), jnp.float32),
            pltpu.VMEM_SHARED((NPAD,), jnp.float32),
            pltpu.SemaphoreType.DMA,
        ],
    )
    def k(col_hbm, out_hbm, colv, ones, zbuf, acc, sem):
        c = lax.axis_index("c")
        t = lax.axis_index("s")
        wid = t * 2 + c
        for i in range(K // 16):
            ones[pl.ds(i * 16, 16)] = jnp.ones((16,), jnp.float32)
        zv = jnp.zeros((16,), jnp.float32)

        def zb(i, _):
            zbuf[pl.ds(i * 16, 16)] = zv
            return 0

        lax.fori_loop(0, seg // 16, zb, 0)
        pltpu.sync_copy(col_hbm.at[pl.ds(wid * nch, nch)], colv)
        off = pl.multiple_of(t * seg, 8)
        pltpu.sync_copy(zbuf, acc.at[pl.ds(off, seg)])
        plsc.subcore_barrier()

        def chunk(g, _):
            cps = [
                pltpu.async_copy(ones, acc.at[colv.at[g * grp + i]], sem, add=True)
                for i in range(grp)
            ]
            for cp in cps:
                cp.wait()
            return 0

        lax.fori_loop(0, nch // grp, chunk, 0)
        plsc.subcore_barrier()
        pltpu.sync_copy(acc.at[pl.ds(off, seg)], zbuf)
        pltpu.sync_copy(zbuf, out_hbm.at[c, pl.ds(off, seg)])

    return k(col2d)


def _agg_sc(y, row2d, col2d):
    nch = ROWS2D // 16
    rpt = HALF // 16

    @functools.partial(
        pl.kernel,
        out_type=jax.ShapeDtypeStruct((NPAD, F), jnp.float32),
        mesh=_mesh(),
        scratch_types=[
            pltpu.VMEM((nch, K), jnp.int32),
            pltpu.VMEM((nch, K), jnp.int32),
            pltpu.VMEM((2, K, F), jnp.float32),
            pltpu.VMEM_SHARED((ACC_ROWS, F), jnp.float32),
            pltpu.SemaphoreType.DMA,
            pltpu.SemaphoreType.DMA,
        ],
    )
    def k(y_hbm, row_hbm, col_hbm, out_hbm, rowv, scatv, gbuf, acc,
          gsem, ssem):
        c = lax.axis_index("c")
        t = lax.axis_index("s")
        pltpu.sync_copy(row_hbm.at[pl.ds(t * nch, nch)], rowv)
        pltpu.sync_copy(col_hbm.at[pl.ds(t * nch, nch)], scatv)
        ybase = pl.multiple_of(c * HALF + t * rpt, 8)
        abase = pl.multiple_of(t * rpt, 8)
        for i in range(rpt // 64):
            pltpu.sync_copy(y_hbm.at[pl.ds(ybase + i * 64, 64)],
                            gbuf.at[0, pl.ds(0, 64)])
            pltpu.sync_copy(gbuf.at[0, pl.ds(0, 64)],
                            acc.at[pl.ds(abase + i * 64, 64)])
        base = c * HALF
        tr = HALF + (lax.iota(jnp.int32, 16) & 7)

        def sidx(j, _):
            for i in range(K // 16):
                col = scatv[j, pl.ds(i * 16, 16)]
                loc = col - base
                ok = (loc >= 0) & (loc < HALF)
                scatv[j, pl.ds(i * 16, 16)] = jnp.where(ok, loc, tr)
            return 0

        lax.fori_loop(0, nch, sidx, 0)
        plsc.subcore_barrier()

        def chunk(j, _):
            pltpu.async_copy(y_hbm.at[rowv.at[j]], gbuf.at[0], gsem).wait()
            pltpu.async_copy(gbuf.at[0], acc.at[scatv.at[j]], ssem,
                             add=True).wait()
            return 0

        lax.fori_loop(0, nch, chunk, 0)
        plsc.subcore_barrier()
        for i in range(rpt // 64):
            pltpu.sync_copy(acc.at[pl.ds(abase + i * 64, 64)],
                            gbuf.at[0, pl.ds(0, 64)])
            pltpu.sync_copy(gbuf.at[0, pl.ds(0, 64)],
                            out_hbm.at[pl.ds(ybase + i * 64, 64)])

    return k(y, row2d, col2d)


BM = 256


def _prescale_tc(parts_t, x_pad):
    def body(p_ref, x_ref, dis_ref, ya_ref, yb_ref):
        p = p_ref[...]
        dis = lax.rsqrt(1.0 + p[:, 0:1] + p[:, 1:2])
        dis_ref[...] = dis
        ya_ref[...] = x_ref[:, :F] * dis
        yb_ref[...] = x_ref[:, F:] * dis

    return pl.pallas_call(
        body,
        grid=(NPAD // BM,),
        in_specs=[pl.BlockSpec((BM, 2), lambda i: (i, 0)),
                  pl.BlockSpec((BM, IN_CH), lambda i: (i, 0))],
        out_specs=[pl.BlockSpec((BM, 1), lambda i: (i, 0)),
                   pl.BlockSpec((BM, F), lambda i: (i, 0)),
                   pl.BlockSpec((BM, F), lambda i: (i, 0))],
        out_shape=[jax.ShapeDtypeStruct((NPAD, 1), jnp.float32),
                   jax.ShapeDtypeStruct((NPAD, F), jnp.float32),
                   jax.ShapeDtypeStruct((NPAD, F), jnp.float32)],
    )(parts_t, x_pad)


def _mm1_tc(agg1a, agg1b, dis, W1, b1):
    BN = 512

    def body(a_ref, b2_ref, d_ref, w_ref, bias_ref, o_ref):
        d = d_ref[...]
        acc = jnp.dot(a_ref[...] * d, w_ref[:F, :],
                      preferred_element_type=jnp.float32)
        acc += jnp.dot(b2_ref[...] * d, w_ref[F:, :],
                       preferred_element_type=jnp.float32)
        o_ref[...] = jnp.maximum(acc + bias_ref[...], 0.0)

    return pl.pallas_call(
        body,
        grid=(NPAD // BM, HID // BN),
        in_specs=[pl.BlockSpec((BM, F), lambda i, j: (i, 0)),
                  pl.BlockSpec((BM, F), lambda i, j: (i, 0)),
                  pl.BlockSpec((BM, 1), lambda i, j: (i, 0)),
                  pl.BlockSpec((IN_CH, BN), lambda i, j: (0, j)),
                  pl.BlockSpec((1, BN), lambda i, j: (0, j))],
        out_specs=pl.BlockSpec((BM, BN), lambda i, j: (i, j)),
        out_shape=jax.ShapeDtypeStruct((NPAD, HID), jnp.float32),
    )(agg1a, agg1b, dis, W1, b1)


def _mm2_tc(h, W2, dis):
    BM2 = 512

    def body(h_ref, w_ref, d_ref, o_ref):
        acc = jnp.dot(h_ref[...], w_ref[...],
                      preferred_element_type=jnp.float32)
        o_ref[...] = acc * d_ref[...]

    return pl.pallas_call(
        body,
        grid=(NPAD // BM2,),
        in_specs=[pl.BlockSpec((BM2, HID), lambda i: (i, 0)),
                  pl.BlockSpec((HID, OUT), lambda i: (0, 0)),
                  pl.BlockSpec((BM2, 1), lambda i: (i, 0))],
        out_specs=pl.BlockSpec((BM2, OUT), lambda i: (i, 0)),
        out_shape=jax.ShapeDtypeStruct((NPAD, OUT), jnp.float32),
    )(h, W2, dis)


def _post_tc(agg2, dis, b2):
    def body(a_ref, d_ref, b_ref, o_ref):
        o_ref[...] = jnp.maximum(a_ref[...] * d_ref[...] + b_ref[...], 0.0)

    return pl.pallas_call(
        body,
        grid=(NPAD // BM,),
        in_specs=[pl.BlockSpec((BM, OUT), lambda i: (i, 0)),
                  pl.BlockSpec((BM, 1), lambda i: (i, 0)),
                  pl.BlockSpec((1, OUT), lambda i: (0, 0))],
        out_specs=pl.BlockSpec((BM, OUT), lambda i: (i, 0)),
        out_shape=jax.ShapeDtypeStruct((NPAD, OUT), jnp.float32),
    )(agg2, dis, b2)


def kernel(x, edge_index, W1, b1, W2, b2):
    pad_i = jnp.arange(E_PAD - E, dtype=jnp.int32)
    rows = jnp.concatenate([edge_index[0], (pad_i * 53) % N])
    cols = jnp.concatenate([edge_index[1], N + (pad_i % 16)])
    row2d = rows.reshape(ROWS2D, K)
    col2d = cols.reshape(ROWS2D, K)
    x_pad = jnp.pad(x, ((0, NPAD - N), (0, 0)))

    parts = _deg_sc(col2d)
    dis, y1a, y1b = _prescale_tc(parts.T, x_pad)
    agg1a = _agg_sc(y1a, row2d, col2d)
    agg1b = _agg_sc(y1b, row2d, col2d)
    h = _mm1_tc(agg1a, agg1b, dis, W1, b1.reshape(1, HID))
    y2 = _mm2_tc(h, W2, dis)
    agg2 = _agg_sc(y2, row2d, col2d)
    z = _post_tc(agg2, dis, b2.reshape(1, OUT))
    return z[:N]


# speedup vs baseline: 15.6975x; 1.7010x over previous
import functools

import jax
import jax.numpy as jnp
from jax import lax
from jax.experimental import pallas as pl
from jax.experimental.pallas import tpu as pltpu
from jax.experimental.pallas import tpu_sc as plsc

N = 10000
E = 160000
IN_CH = 256
HID = 1024
OUT = 128

NPAD = 10240
HALF = 5120
ACC_ROWS = HALF + 16
E_PAD = 163840
K = 128
ROWS2D = E_PAD // K
F = 128


def _mesh():
    return plsc.VectorSubcoreMesh(core_axis_name="c", subcore_axis_name="s")


def _deg_sc(col2d):
    nch = ROWS2D // 32
    seg = NPAD // 16
    grp = 8

    @functools.partial(
        pl.kernel,
        out_type=jax.ShapeDtypeStruct((2, NPAD), jnp.float32),
        mesh=_mesh(),
        scratch_types=[
            pltpu.VMEM((nch, K), jnp.int32),
            pltpu.VMEM((K,), jnp.float32),
            pltpu.VMEM((seg,), jnp.float32),
            pltpu.VMEM_SHARED((NPAD,), jnp.float32),
            pltpu.SemaphoreType.DMA,
        ],
    )
    def k(col_hbm, out_hbm, colv, ones, zbuf, acc, sem):
        c = lax.axis_index("c")
        t = lax.axis_index("s")
        wid = t * 2 + c
        for i in range(K // 16):
            ones[pl.ds(i * 16, 16)] = jnp.ones((16,), jnp.float32)
        zv = jnp.zeros((16,), jnp.float32)

        def zb(i, _):
            zbuf[pl.ds(i * 16, 16)] = zv
            return 0

        lax.fori_loop(0, seg // 16, zb, 0)
        pltpu.sync_copy(col_hbm.at[pl.ds(wid * nch, nch)], colv)
        off = pl.multiple_of(t * seg, 8)
        pltpu.sync_copy(zbuf, acc.at[pl.ds(off, seg)])
        plsc.subcore_barrier()

        def chunk(g, _):
            cps = [
                pltpu.async_copy(ones, acc.at[colv.at[g * grp + i]], sem, add=True)
                for i in range(grp)
            ]
            for cp in cps:
                cp.wait()
            return 0

        lax.fori_loop(0, nch // grp, chunk, 0)
        plsc.subcore_barrier()
        pltpu.sync_copy(acc.at[pl.ds(off, seg)], zbuf)
        pltpu.sync_copy(zbuf, out_hbm.at[c, pl.ds(off, seg)])

    return k(col2d)


def _agg_sc(y, row2d, col2d):
    nch = ROWS2D // 32
    rpt = NPAD // 16

    @functools.partial(
        pl.kernel,
        out_type=jax.ShapeDtypeStruct((2 * NPAD, F), jnp.float32),
        mesh=_mesh(),
        scratch_types=[
            pltpu.VMEM((nch, K), jnp.int32),
            pltpu.VMEM((nch, K), jnp.int32),
            pltpu.VMEM((2, K, F), jnp.float32),
            pltpu.VMEM_SHARED((NPAD, F), jnp.float32),
            pltpu.SemaphoreType.DMA,
            pltpu.SemaphoreType.DMA,
        ],
    )
    def k(y_hbm, row_hbm, col_hbm, out_hbm, rowv, colv, gbuf, acc,
          gsem0, gsem1):
        c = lax.axis_index("c")
        t = lax.axis_index("s")
        wid = t * 2 + c
        pltpu.sync_copy(row_hbm.at[pl.ds(wid * nch, nch)], rowv)
        pltpu.sync_copy(col_hbm.at[pl.ds(wid * nch, nch)], colv)
        abase = pl.multiple_of(t * rpt, 8)
        for i in range(rpt // 64):
            pltpu.sync_copy(y_hbm.at[pl.ds(abase + i * 64, 64)],
                            gbuf.at[0, pl.ds(0, 64)])
            pltpu.sync_copy(gbuf.at[0, pl.ds(0, 64)],
                            acc.at[pl.ds(abase + i * 64, 64)])
        plsc.subcore_barrier()

        def start_gather(j, b, sem):
            pltpu.async_copy(y_hbm.at[rowv.at[j]], gbuf.at[b], sem)

        def wait_gather(j, b, sem):
            pltpu.make_async_copy(y_hbm.at[rowv.at[j]], gbuf.at[b],
                                  sem).wait()

        def scat(j, b):
            pltpu.sync_copy(gbuf.at[b], acc.at[colv.at[j]], add=True)

        start_gather(0, 0, gsem0)

        def pair(g, _):
            j0 = 2 * g
            start_gather(j0 + 1, 1, gsem1)
            wait_gather(j0, 0, gsem0)
            scat(j0, 0)
            start_gather(j0 + 2, 0, gsem0)
            wait_gather(j0 + 1, 1, gsem1)
            scat(j0 + 1, 1)
            return 0

        lax.fori_loop(0, nch // 2 - 1, pair, 0)
        start_gather(nch - 1, 1, gsem1)
        wait_gather(nch - 2, 0, gsem0)
        scat(nch - 2, 0)
        wait_gather(nch - 1, 1, gsem1)
        scat(nch - 1, 1)
        plsc.subcore_barrier()
        obase = pl.multiple_of(c * NPAD + t * rpt, 8)
        for i in range(rpt // 64):
            pltpu.sync_copy(acc.at[pl.ds(abase + i * 64, 64)],
                            gbuf.at[0, pl.ds(0, 64)])
            pltpu.sync_copy(gbuf.at[0, pl.ds(0, 64)],
                            out_hbm.at[pl.ds(obase + i * 64, 64)])

    return k(y, row2d, col2d)


BM = 256


def _prescale_tc(parts_t, x_pad):
    def body(p_ref, x_ref, dis_ref, ya_ref, yb_ref):
        p = p_ref[...]
        dis = lax.rsqrt(1.0 + p[:, 0:1] + p[:, 1:2])
        dis_ref[...] = dis
        ya_ref[...] = x_ref[:, :F] * dis
        yb_ref[...] = x_ref[:, F:] * dis

    return pl.pallas_call(
        body,
        grid=(NPAD // BM,),
        in_specs=[pl.BlockSpec((BM, 2), lambda i: (i, 0)),
                  pl.BlockSpec((BM, IN_CH), lambda i: (i, 0))],
        out_specs=[pl.BlockSpec((BM, 1), lambda i: (i, 0)),
                   pl.BlockSpec((BM, F), lambda i: (i, 0)),
                   pl.BlockSpec((BM, F), lambda i: (i, 0))],
        out_shape=[jax.ShapeDtypeStruct((NPAD, 1), jnp.float32),
                   jax.ShapeDtypeStruct((NPAD, F), jnp.float32),
                   jax.ShapeDtypeStruct((NPAD, F), jnp.float32)],
    )(parts_t, x_pad)


def _mm1_tc(Pa, Pb, y1a, y1b, dis, W1, b1):
    BN = 512
    NB = NPAD // BM

    def body(p0a, p1a, ya, p0b, p1b, yb, d_ref, w_ref, bias_ref, o_ref):
        d = d_ref[...]
        a = (p0a[...] + p1a[...] - ya[...]) * d
        b = (p0b[...] + p1b[...] - yb[...]) * d
        acc = jnp.dot(a, w_ref[:F, :], preferred_element_type=jnp.float32)
        acc += jnp.dot(b, w_ref[F:, :], preferred_element_type=jnp.float32)
        o_ref[...] = jnp.maximum(acc + bias_ref[...], 0.0)

    return pl.pallas_call(
        body,
        grid=(NB, HID // BN),
        in_specs=[pl.BlockSpec((BM, F), lambda i, j: (i, 0)),
                  pl.BlockSpec((BM, F), lambda i, j: (i + NB, 0)),
                  pl.BlockSpec((BM, F), lambda i, j: (i, 0)),
                  pl.BlockSpec((BM, F), lambda i, j: (i, 0)),
                  pl.BlockSpec((BM, F), lambda i, j: (i + NB, 0)),
                  pl.BlockSpec((BM, F), lambda i, j: (i, 0)),
                  pl.BlockSpec((BM, 1), lambda i, j: (i, 0)),
                  pl.BlockSpec((IN_CH, BN), lambda i, j: (0, j)),
                  pl.BlockSpec((1, BN), lambda i, j: (0, j))],
        out_specs=pl.BlockSpec((BM, BN), lambda i, j: (i, j)),
        out_shape=jax.ShapeDtypeStruct((NPAD, HID), jnp.float32),
    )(Pa, Pa, y1a, Pb, Pb, y1b, dis, W1, b1)


def _mm2_tc(h, W2, dis):
    BM2 = 512

    def body(h_ref, w_ref, d_ref, o_ref):
        acc = jnp.dot(h_ref[...], w_ref[...],
                      preferred_element_type=jnp.float32)
        o_ref[...] = acc * d_ref[...]

    return pl.pallas_call(
        body,
        grid=(NPAD // BM2,),
        in_specs=[pl.BlockSpec((BM2, HID), lambda i: (i, 0)),
                  pl.BlockSpec((HID, OUT), lambda i: (0, 0)),
                  pl.BlockSpec((BM2, 1), lambda i: (i, 0))],
        out_specs=pl.BlockSpec((BM2, OUT), lambda i: (i, 0)),
        out_shape=jax.ShapeDtypeStruct((NPAD, OUT), jnp.float32),
    )(h, W2, dis)


def _post_tc(P2, y2, dis, b2):
    NB = NPAD // BM

    def body(q0, q1, y_ref, d_ref, b_ref, o_ref):
        a = q0[...] + q1[...] - y_ref[...]
        o_ref[...] = jnp.maximum(a * d_ref[...] + b_ref[...], 0.0)

    return pl.pallas_call(
        body,
        grid=(NB,),
        in_specs=[pl.BlockSpec((BM, OUT), lambda i: (i, 0)),
                  pl.BlockSpec((BM, OUT), lambda i: (i + NB, 0)),
                  pl.BlockSpec((BM, OUT), lambda i: (i, 0)),
                  pl.BlockSpec((BM, 1), lambda i: (i, 0)),
                  pl.BlockSpec((1, OUT), lambda i: (0, 0))],
        out_specs=pl.BlockSpec((BM, OUT), lambda i: (i, 0)),
        out_shape=jax.ShapeDtypeStruct((NPAD, OUT), jnp.float32),
    )(P2, P2, y2, dis, b2)


def kernel(x, edge_index, W1, b1, W2, b2):
    pad_i = jnp.arange(E_PAD - E, dtype=jnp.int32)
    rows = jnp.concatenate([edge_index[0], (pad_i * 53) % N])
    cols = jnp.concatenate([edge_index[1], N + (pad_i % 16)])
    row2d = rows.reshape(ROWS2D, K)
    col2d = cols.reshape(ROWS2D, K)
    x_pad = jnp.pad(x, ((0, NPAD - N), (0, 0)))

    parts = _deg_sc(col2d)
    dis, y1a, y1b = _prescale_tc(parts.T, x_pad)
    Pa = _agg_sc(y1a, row2d, col2d)
    Pb = _agg_sc(y1b, row2d, col2d)
    h = _mm1_tc(Pa, Pb, y1a, y1b, dis, W1, b1.reshape(1, HID))
    y2 = _mm2_tc(h, W2, dis)
    P2 = _agg_sc(y2, row2d, col2d)
    z = _post_tc(P2, y2, dis, b2.reshape(1, OUT))
    return z[:N]


# speedup vs baseline: 16.7791x; 1.0689x over previous
import functools

import jax
import jax.numpy as jnp
from jax import lax
from jax.experimental import pallas as pl
from jax.experimental.pallas import tpu as pltpu
from jax.experimental.pallas import tpu_sc as plsc

N = 10000
E = 160000
IN_CH = 256
HID = 1024
OUT = 128

NPAD = 10240
HALF = 5120
ACC_ROWS = HALF + 16
E_PAD = 163840
K = 128
ROWS2D = E_PAD // K
F = 128


def _mesh():
    return plsc.VectorSubcoreMesh(core_axis_name="c", subcore_axis_name="s")


def _deg_sc(col2d):
    nch = ROWS2D // 32
    seg = NPAD // 16
    grp = 8

    @functools.partial(
        pl.kernel,
        out_type=jax.ShapeDtypeStruct((2, NPAD), jnp.float32),
        mesh=_mesh(),
        scratch_types=[
            pltpu.VMEM((nch, K), jnp.int32),
            pltpu.VMEM((K,), jnp.float32),
            pltpu.VMEM((seg,), jnp.float32),
            pltpu.VMEM_SHARED((NPAD,), jnp.float32),
            pltpu.SemaphoreType.DMA,
        ],
    )
    def k(col_hbm, out_hbm, colv, ones, zbuf, acc, sem):
        c = lax.axis_index("c")
        t = lax.axis_index("s")
        wid = t * 2 + c
        for i in range(K // 16):
            ones[pl.ds(i * 16, 16)] = jnp.ones((16,), jnp.float32)
        zv = jnp.zeros((16,), jnp.float32)

        def zb(i, _):
            zbuf[pl.ds(i * 16, 16)] = zv
            return 0

        lax.fori_loop(0, seg // 16, zb, 0)
        pltpu.sync_copy(col_hbm.at[pl.ds(wid * nch, nch)], colv)
        off = pl.multiple_of(t * seg, 8)
        pltpu.sync_copy(zbuf, acc.at[pl.ds(off, seg)])
        plsc.subcore_barrier()

        def chunk(g, _):
            cps = [
                pltpu.async_copy(ones, acc.at[colv.at[g * grp + i]], sem, add=True)
                for i in range(grp)
            ]
            for cp in cps:
                cp.wait()
            return 0

        lax.fori_loop(0, nch // grp, chunk, 0)
        plsc.subcore_barrier()
        pltpu.sync_copy(acc.at[pl.ds(off, seg)], zbuf)
        pltpu.sync_copy(zbuf, out_hbm.at[c, pl.ds(off, seg)])

    return k(col2d)


def _agg_sc(ys, row2d, col2d):
    nch = ROWS2D // 32
    rpt = NPAD // 16
    ny = len(ys)

    @functools.partial(
        pl.kernel,
        out_type=[jax.ShapeDtypeStruct((2 * NPAD, F), jnp.float32)] * ny,
        mesh=_mesh(),
        scratch_types=[
            pltpu.VMEM((nch, K), jnp.int32),
            pltpu.VMEM((nch, K), jnp.int32),
            pltpu.VMEM((2, K, F), jnp.float32),
            pltpu.VMEM_SHARED((NPAD, F), jnp.float32),
            pltpu.SemaphoreType.DMA,
            pltpu.SemaphoreType.DMA,
        ],
    )
    def k(*args):
        y_hbms = args[:ny]
        row_hbm, col_hbm = args[ny], args[ny + 1]
        out_hbms = args[ny + 2:2 * ny + 2]
        rowv, colv, gbuf, acc, gsem0, gsem1 = args[2 * ny + 2:]
        c = lax.axis_index("c")
        t = lax.axis_index("s")
        wid = t * 2 + c
        pltpu.sync_copy(row_hbm.at[pl.ds(wid * nch, nch)], rowv)
        pltpu.sync_copy(col_hbm.at[pl.ds(wid * nch, nch)], colv)
        abase = pl.multiple_of(t * rpt, 8)
        obase = pl.multiple_of(c * NPAD + t * rpt, 8)

        for y_hbm, out_hbm in zip(y_hbms, out_hbms):
            pltpu.sync_copy(y_hbm.at[pl.ds(abase, rpt)],
                            acc.at[pl.ds(abase, rpt)])
            plsc.subcore_barrier()

            def start_gather(j, b, sem):
                pltpu.async_copy(y_hbm.at[rowv.at[j]], gbuf.at[b], sem)

            def wait_gather(j, b, sem):
                pltpu.make_async_copy(y_hbm.at[rowv.at[j]], gbuf.at[b],
                                      sem).wait()

            def scat(j, b):
                pltpu.sync_copy(gbuf.at[b], acc.at[colv.at[j]], add=True)

            start_gather(0, 0, gsem0)

            def pair(g, _):
                j0 = 2 * g
                start_gather(j0 + 1, 1, gsem1)
                wait_gather(j0, 0, gsem0)
                scat(j0, 0)
                start_gather(j0 + 2, 0, gsem0)
                wait_gather(j0 + 1, 1, gsem1)
                scat(j0 + 1, 1)
                return 0

            lax.fori_loop(0, nch // 2 - 1, pair, 0)
            start_gather(nch - 1, 1, gsem1)
            wait_gather(nch - 2, 0, gsem0)
            scat(nch - 2, 0)
            wait_gather(nch - 1, 1, gsem1)
            scat(nch - 1, 1)
            plsc.subcore_barrier()
            pltpu.sync_copy(acc.at[pl.ds(abase, rpt)],
                            out_hbm.at[pl.ds(obase, rpt)])
            plsc.subcore_barrier()

    outs = k(*ys, row2d, col2d)
    return outs if ny > 1 else outs


BM = 256


def _prescale_tc(parts_t, x_pad):
    def body(p_ref, x_ref, dis_ref, ya_ref, yb_ref):
        p = p_ref[...]
        dis = lax.rsqrt(1.0 + p[:, 0:1] + p[:, 1:2])
        dis_ref[...] = dis
        ya_ref[...] = x_ref[:, :F] * dis
        yb_ref[...] = x_ref[:, F:] * dis

    return pl.pallas_call(
        body,
        grid=(NPAD // BM,),
        in_specs=[pl.BlockSpec((BM, 2), lambda i: (i, 0)),
                  pl.BlockSpec((BM, IN_CH), lambda i: (i, 0))],
        out_specs=[pl.BlockSpec((BM, 1), lambda i: (i, 0)),
                   pl.BlockSpec((BM, F), lambda i: (i, 0)),
                   pl.BlockSpec((BM, F), lambda i: (i, 0))],
        out_shape=[jax.ShapeDtypeStruct((NPAD, 1), jnp.float32),
                   jax.ShapeDtypeStruct((NPAD, F), jnp.float32),
                   jax.ShapeDtypeStruct((NPAD, F), jnp.float32)],
    )(parts_t, x_pad)


def _mm1_tc(Pa, Pb, y1a, y1b, dis, W1, b1):
    BN = 512
    NB = NPAD // BM

    def body(p0a, p1a, ya, p0b, p1b, yb, d_ref, w_ref, bias_ref, o_ref):
        d = d_ref[...]
        a = (p0a[...] + p1a[...] - ya[...]) * d
        b = (p0b[...] + p1b[...] - yb[...]) * d
        acc = jnp.dot(a, w_ref[:F, :], preferred_element_type=jnp.float32)
        acc += jnp.dot(b, w_ref[F:, :], preferred_element_type=jnp.float32)
        o_ref[...] = jnp.maximum(acc + bias_ref[...], 0.0)

    return pl.pallas_call(
        body,
        grid=(NB, HID // BN),
        in_specs=[pl.BlockSpec((BM, F), lambda i, j: (i, 0)),
                  pl.BlockSpec((BM, F), lambda i, j: (i + NB, 0)),
                  pl.BlockSpec((BM, F), lambda i, j: (i, 0)),
                  pl.BlockSpec((BM, F), lambda i, j: (i, 0)),
                  pl.BlockSpec((BM, F), lambda i, j: (i + NB, 0)),
                  pl.BlockSpec((BM, F), lambda i, j: (i, 0)),
                  pl.BlockSpec((BM, 1), lambda i, j: (i, 0)),
                  pl.BlockSpec((IN_CH, BN), lambda i, j: (0, j)),
                  pl.BlockSpec((1, BN), lambda i, j: (0, j))],
        out_specs=pl.BlockSpec((BM, BN), lambda i, j: (i, j)),
        out_shape=jax.ShapeDtypeStruct((NPAD, HID), jnp.float32),
    )(Pa, Pa, y1a, Pb, Pb, y1b, dis, W1, b1)


def _mm2_tc(h, W2, dis):
    BM2 = 512

    def body(h_ref, w_ref, d_ref, o_ref):
        acc = jnp.dot(h_ref[...], w_ref[...],
                      preferred_element_type=jnp.float32)
        o_ref[...] = acc * d_ref[...]

    return pl.pallas_call(
        body,
        grid=(NPAD // BM2,),
        in_specs=[pl.BlockSpec((BM2, HID), lambda i: (i, 0)),
                  pl.BlockSpec((HID, OUT), lambda i: (0, 0)),
                  pl.BlockSpec((BM2, 1), lambda i: (i, 0))],
        out_specs=pl.BlockSpec((BM2, OUT), lambda i: (i, 0)),
        out_shape=jax.ShapeDtypeStruct((NPAD, OUT), jnp.float32),
    )(h, W2, dis)


def _post_tc(P2, y2, dis, b2):
    NB = NPAD // BM

    def body(q0, q1, y_ref, d_ref, b_ref, o_ref):
        a = q0[...] + q1[...] - y_ref[...]
        o_ref[...] = jnp.maximum(a * d_ref[...] + b_ref[...], 0.0)

    return pl.pallas_call(
        body,
        grid=(NB,),
        in_specs=[pl.BlockSpec((BM, OUT), lambda i: (i, 0)),
                  pl.BlockSpec((BM, OUT), lambda i: (i + NB, 0)),
                  pl.BlockSpec((BM, OUT), lambda i: (i, 0)),
                  pl.BlockSpec((BM, 1), lambda i: (i, 0)),
                  pl.BlockSpec((1, OUT), lambda i: (0, 0))],
        out_specs=pl.BlockSpec((BM, OUT), lambda i: (i, 0)),
        out_shape=jax.ShapeDtypeStruct((NPAD, OUT), jnp.float32),
    )(P2, P2, y2, dis, b2)


def kernel(x, edge_index, W1, b1, W2, b2):
    pad_i = jnp.arange(E_PAD - E, dtype=jnp.int32)
    rows = jnp.concatenate([edge_index[0], (pad_i * 53) % N])
    cols = jnp.concatenate([edge_index[1], N + (pad_i % 16)])
    row2d = rows.reshape(ROWS2D, K)
    col2d = cols.reshape(ROWS2D, K)
    x_pad = jnp.pad(x, ((0, NPAD - N), (0, 0)))

    parts = _deg_sc(col2d)
    dis, y1a, y1b = _prescale_tc(parts.T, x_pad)
    Pa, Pb = _agg_sc([y1a, y1b], row2d, col2d)
    h = _mm1_tc(Pa, Pb, y1a, y1b, dis, W1, b1.reshape(1, HID))
    y2 = _mm2_tc(h, W2, dis)
    (P2,) = _agg_sc([y2], row2d, col2d)
    z = _post_tc(P2, y2, dis, b2.reshape(1, OUT))
    return z[:N]


# speedup vs baseline: 20.3866x; 1.2150x over previous
import functools

import jax
import jax.numpy as jnp
from jax import lax
from jax.experimental import pallas as pl
from jax.experimental.pallas import tpu as pltpu
from jax.experimental.pallas import tpu_sc as plsc

N = 10000
E = 160000
IN_CH = 256
HID = 1024
OUT = 128

NPAD = 10240
HALF = 5120
ACC_ROWS = HALF + 16
E_PAD = 163840
K = 128
ROWS2D = E_PAD // K
F = 128


def _mesh():
    return plsc.VectorSubcoreMesh(core_axis_name="c", subcore_axis_name="s")


def _deg_sc(col2d):
    nch = ROWS2D // 32
    seg = NPAD // 16
    grp = 8

    @functools.partial(
        pl.kernel,
        out_type=jax.ShapeDtypeStruct((2, NPAD), jnp.float32),
        mesh=_mesh(),
        scratch_types=[
            pltpu.VMEM((nch, K), jnp.int32),
            pltpu.VMEM((K,), jnp.float32),
            pltpu.VMEM((seg,), jnp.float32),
            pltpu.VMEM_SHARED((NPAD,), jnp.float32),
            pltpu.SemaphoreType.DMA,
        ],
    )
    def k(col_hbm, out_hbm, colv, ones, zbuf, acc, sem):
        c = lax.axis_index("c")
        t = lax.axis_index("s")
        wid = t * 2 + c
        for i in range(K // 16):
            ones[pl.ds(i * 16, 16)] = jnp.ones((16,), jnp.float32)
        zv = jnp.zeros((16,), jnp.float32)

        def zb(i, _):
            zbuf[pl.ds(i * 16, 16)] = zv
            return 0

        lax.fori_loop(0, seg // 16, zb, 0)
        pltpu.sync_copy(col_hbm.at[pl.ds(wid * nch, nch)], colv)
        off = pl.multiple_of(t * seg, 8)
        pltpu.sync_copy(zbuf, acc.at[pl.ds(off, seg)])
        plsc.subcore_barrier()

        def chunk(g, _):
            cps = [
                pltpu.async_copy(ones, acc.at[colv.at[g * grp + i]], sem, add=True)
                for i in range(grp)
            ]
            for cp in cps:
                cp.wait()
            return 0

        lax.fori_loop(0, nch // grp, chunk, 0)
        plsc.subcore_barrier()
        pltpu.sync_copy(acc.at[pl.ds(off, seg)], zbuf)
        pltpu.sync_copy(zbuf, out_hbm.at[c, pl.ds(off, seg)])

    return k(col2d)


def _agg_sc(ys, row2d, col2d):
    nch = ROWS2D // 32
    rpt = NPAD // 16
    ny = len(ys)

    @functools.partial(
        pl.kernel,
        out_type=[jax.ShapeDtypeStruct((2 * NPAD, F), jnp.float32)] * ny,
        mesh=_mesh(),
        scratch_types=[
            pltpu.VMEM((nch, K), jnp.int32),
            pltpu.VMEM((nch, K), jnp.int32),
            pltpu.VMEM((2, K, F), jnp.float32),
            pltpu.VMEM_SHARED((NPAD, F), jnp.float32),
            pltpu.SemaphoreType.DMA,
            pltpu.SemaphoreType.DMA,
        ],
    )
    def k(*args):
        y_hbms = args[:ny]
        row_hbm, col_hbm = args[ny], args[ny + 1]
        out_hbms = args[ny + 2:2 * ny + 2]
        rowv, colv, gbuf, acc, gsem0, gsem1 = args[2 * ny + 2:]
        c = lax.axis_index("c")
        t = lax.axis_index("s")
        wid = t * 2 + c
        pltpu.sync_copy(row_hbm.at[pl.ds(wid * nch, nch)], rowv)
        pltpu.sync_copy(col_hbm.at[pl.ds(wid * nch, nch)], colv)
        abase = pl.multiple_of(t * rpt, 8)
        obase = pl.multiple_of(c * NPAD + t * rpt, 8)

        for y_hbm, out_hbm in zip(y_hbms, out_hbms):
            pltpu.sync_copy(y_hbm.at[pl.ds(abase, rpt)],
                            acc.at[pl.ds(abase, rpt)])
            plsc.subcore_barrier()

            def start_gather(j, b, sem):
                pltpu.async_copy(y_hbm.at[rowv.at[j]], gbuf.at[b], sem)

            def wait_gather(j, b, sem):
                pltpu.make_async_copy(y_hbm.at[rowv.at[j]], gbuf.at[b],
                                      sem).wait()

            def scat(j, b):
                pltpu.sync_copy(gbuf.at[b], acc.at[colv.at[j]], add=True)

            start_gather(0, 0, gsem0)

            def pair(g, _):
                j0 = 2 * g
                start_gather(j0 + 1, 1, gsem1)
                wait_gather(j0, 0, gsem0)
                scat(j0, 0)
                start_gather(j0 + 2, 0, gsem0)
                wait_gather(j0 + 1, 1, gsem1)
                scat(j0 + 1, 1)
                return 0

            lax.fori_loop(0, nch // 2 - 1, pair, 0)
            start_gather(nch - 1, 1, gsem1)
            wait_gather(nch - 2, 0, gsem0)
            scat(nch - 2, 0)
            wait_gather(nch - 1, 1, gsem1)
            scat(nch - 1, 1)
            plsc.subcore_barrier()
            pltpu.sync_copy(acc.at[pl.ds(abase, rpt)],
                            out_hbm.at[pl.ds(obase, rpt)])
            plsc.subcore_barrier()

    outs = k(*ys, row2d, col2d)
    return outs if ny > 1 else outs


BM = 256


def _prescale_tc(parts_t, x_pad):
    def body(p_ref, x_ref, dis_ref, ya_ref, yb_ref):
        p = p_ref[...]
        dis = lax.rsqrt(1.0 + p[:, 0:1] + p[:, 1:2])
        dis_ref[...] = dis
        ya_ref[...] = x_ref[:, :F] * dis
        yb_ref[...] = x_ref[:, F:] * dis

    return pl.pallas_call(
        body,
        grid=(NPAD // BM,),
        in_specs=[pl.BlockSpec((BM, 2), lambda i: (i, 0)),
                  pl.BlockSpec((BM, IN_CH), lambda i: (i, 0))],
        out_specs=[pl.BlockSpec((BM, 1), lambda i: (i, 0)),
                   pl.BlockSpec((BM, F), lambda i: (i, 0)),
                   pl.BlockSpec((BM, F), lambda i: (i, 0))],
        out_shape=[jax.ShapeDtypeStruct((NPAD, 1), jnp.float32),
                   jax.ShapeDtypeStruct((NPAD, F), jnp.float32),
                   jax.ShapeDtypeStruct((NPAD, F), jnp.float32)],
    )(parts_t, x_pad)


def _mlp_tc(Pa, Pb, y1a, y1b, dis, W1, b1, W2):
    NB = NPAD // BM

    def body(p0a, p1a, ya, p0b, p1b, yb, d_ref, w1_ref, bias_ref, w2_ref,
             o_ref):
        d = d_ref[...]
        a = (p0a[...] + p1a[...] - ya[...]) * d
        b = (p0b[...] + p1b[...] - yb[...]) * d
        acc = jnp.dot(a, w1_ref[:F, :], preferred_element_type=jnp.float32)
        acc += jnp.dot(b, w1_ref[F:, :], preferred_element_type=jnp.float32)
        h = jnp.maximum(acc + bias_ref[...], 0.0)
        o_ref[...] = jnp.dot(h, w2_ref[...],
                             preferred_element_type=jnp.float32) * d

    return pl.pallas_call(
        body,
        grid=(NB,),
        in_specs=[pl.BlockSpec((BM, F), lambda i: (i, 0)),
                  pl.BlockSpec((BM, F), lambda i: (i + NB, 0)),
                  pl.BlockSpec((BM, F), lambda i: (i, 0)),
                  pl.BlockSpec((BM, F), lambda i: (i, 0)),
                  pl.BlockSpec((BM, F), lambda i: (i + NB, 0)),
                  pl.BlockSpec((BM, F), lambda i: (i, 0)),
                  pl.BlockSpec((BM, 1), lambda i: (i, 0)),
                  pl.BlockSpec((IN_CH, HID), lambda i: (0, 0)),
                  pl.BlockSpec((1, HID), lambda i: (0, 0)),
                  pl.BlockSpec((HID, OUT), lambda i: (0, 0))],
        out_specs=pl.BlockSpec((BM, OUT), lambda i: (i, 0)),
        out_shape=jax.ShapeDtypeStruct((NPAD, OUT), jnp.float32),
    )(Pa, Pa, y1a, Pb, Pb, y1b, dis, W1, b1, W2)


def _post_tc(P2, y2, dis, b2):
    NB = NPAD // BM

    def body(q0, q1, y_ref, d_ref, b_ref, o_ref):
        a = q0[...] + q1[...] - y_ref[...]
        o_ref[...] = jnp.maximum(a * d_ref[...] + b_ref[...], 0.0)

    return pl.pallas_call(
        body,
        grid=(NB,),
        in_specs=[pl.BlockSpec((BM, OUT), lambda i: (i, 0)),
                  pl.BlockSpec((BM, OUT), lambda i: (i + NB, 0)),
                  pl.BlockSpec((BM, OUT), lambda i: (i, 0)),
                  pl.BlockSpec((BM, 1), lambda i: (i, 0)),
                  pl.BlockSpec((1, OUT), lambda i: (0, 0))],
        out_specs=pl.BlockSpec((BM, OUT), lambda i: (i, 0)),
        out_shape=jax.ShapeDtypeStruct((NPAD, OUT), jnp.float32),
    )(P2, P2, y2, dis, b2)


def kernel(x, edge_index, W1, b1, W2, b2):
    pad_i = jnp.arange(E_PAD - E, dtype=jnp.int32)
    rows = jnp.concatenate([edge_index[0], (pad_i * 53) % N])
    cols = jnp.concatenate([edge_index[1], N + (pad_i % 16)])
    row2d = rows.reshape(ROWS2D, K)
    col2d = cols.reshape(ROWS2D, K)
    x_pad = jnp.pad(x, ((0, NPAD - N), (0, 0)))

    parts = _deg_sc(col2d)
    dis, y1a, y1b = _prescale_tc(parts.T, x_pad)
    Pa, Pb = _agg_sc([y1a, y1b], row2d, col2d)
    y2 = _mlp_tc(Pa, Pb, y1a, y1b, dis, W1, b1.reshape(1, HID), W2)
    (P2,) = _agg_sc([y2], row2d, col2d)
    z = _post_tc(P2, y2, dis, b2.reshape(1, OUT))
    return z[:N]
